# baseline (speedup 1.0000x reference)
import functools

import jax
import jax.numpy as jnp
from jax import lax
from jax.experimental import pallas as pl
from jax.experimental.pallas import tpu as pltpu
from jax.experimental.pallas import tpu_sc as plsc

NV = 26
DEPTH = 32
DE = 7
DM = 128
B = 1024
NTOK = NV * B * DEPTH
_LR = 8
NROWS = _LR * NV

_NC = 2
_NS = 16
_NW = _NC * _NS
_RPW = NV * B // _NW
_RS = 8
_S = _RS * DEPTH
_NG = 2
_SG = _S // _NG
_STEPS = _RPW // _RS
_NB = 2


def _lut_body(emb_ref, w_ref, b_ref, g_ref, bt_ref, lut_ref):
    rowmask = (lax.broadcasted_iota(jnp.int32, (_LR, 1), 0) < 2).astype(jnp.float32)
    for i in range(NV):
        e = jnp.concatenate(
            [emb_ref[i], jnp.zeros((_LR - 3, DE), jnp.float32)]) * rowmask
        h = lax.dot_general(e, w_ref[i], (((1,), (0,)), ((), ())),
                            preferred_element_type=jnp.float32)
        h = h + b_ref[i]
        mu = jnp.mean(h, axis=-1, keepdims=True)
        var = jnp.mean((h - mu) ** 2, axis=-1, keepdims=True)
        lut_ref[i] = (h - mu) * lax.rsqrt(var + 1e-5) * g_ref[i] + bt_ref[i]


def _lut(emb_tables, W, b3, g3, bt3):
    return pl.pallas_call(
        _lut_body,
        out_shape=jax.ShapeDtypeStruct((NV, _LR, DM), jnp.float32),
    )(emb_tables, W, b3, g3, bt3)


def _sc_body(lut_hbm, x_hbm, out_hbm, lut_s,
             x_v, idx_v, rows_v, sem_x, sem_g, sem_o):
    sid = lax.axis_index("s")
    wid = sid * _NC + lax.axis_index("c")
    base_r = wid * _RPW

    @pl.when(sid == 0)
    def _():
        pltpu.sync_copy(lut_hbm, lut_s)
    plsc.subcore_barrier()

    def fire_x(s, j):
        r0 = base_r + s * _RS
        pltpu.async_copy(x_hbm.at[pl.ds(r0 % B, _RS), pl.ds(r0 // B, 1)],
                         x_v[j], sem_x[j])

    def wait_x(j):
        pltpu.make_async_copy(x_hbm.at[pl.ds(0, _RS), pl.ds(0, 1)],
                              x_v[j], sem_x[j]).wait()

    def wait_o(j):
        pltpu.make_async_copy(rows_v[j], out_hbm.at[pl.ds(0, _S)],
                              sem_o[j]).wait()

    for j in range(_NB):
        fire_x(j, j)

    def step(k, carry):
        descs = []
        for j in range(_NB):
            s = k * _NB + j
            r0 = base_r + s * _RS
            rowb = _LR * (r0 // B)
            wait_x(j)

            @pl.when(k >= 1)
            def _():
                wait_o(j)

            for g in range(_S // 16):
                xv = x_v[j][g // 2, 0, pl.ds((g % 2) * 16, 16)]
                c = jnp.where(xv < 255, jnp.minimum(xv, 1), 2)
                idx_v[j][g // 8, pl.ds((g % 8) * 16, 16)] = rowb + c

            for q in range(_NG):
                descs.append(pltpu.async_copy(
                    lut_s.at[idx_v[j].at[q]],
                    rows_v[j].at[pl.ds(q * _SG, _SG)], sem_g[j]))

            @pl.when(k < (_STEPS // _NB) - 1)
            def _():
                fire_x(s + _NB, j)

        for j in range(_NB):
            s = k * _NB + j
            r0 = base_r + s * _RS
            for q in range(_NG):
                descs[j * _NG + q].wait()
            pltpu.async_copy(rows_v[j], out_hbm.at[pl.ds(r0 * DEPTH, _S)],
                             sem_o[j])
        return carry

    lax.fori_loop(0, _STEPS // _NB, step, 0)
    for j in range(_NB):
        wait_o(j)


@functools.cache
def _sc_expand():
    return functools.partial(
        pl.kernel,
        out_type=jax.ShapeDtypeStruct((NTOK, DM), jnp.float32),
        mesh=plsc.VectorSubcoreMesh(core_axis_name="c", subcore_axis_name="s"),
        scratch_types=[
            pltpu.VMEM_SHARED((NROWS, DM), jnp.float32),
            [pltpu.VMEM((_RS, 1, DEPTH), jnp.int32) for _ in range(_NB)],
            [pltpu.VMEM((_NG, _SG), jnp.int32) for _ in range(_NB)],
            [pltpu.VMEM((_S, DM), jnp.float32) for _ in range(_NB)],
            [pltpu.SemaphoreType.DMA for _ in range(_NB)],
            [pltpu.SemaphoreType.DMA for _ in range(_NB)],
            [pltpu.SemaphoreType.DMA for _ in range(_NB)],
        ],
    )(_sc_body)


_BBLK = 32


def _tc_expand_body(x_ref, lut_ref, out_ref):
    for i in range(NV):
        xi = x_ref[:, i, :]
        f0 = (xi == 0).astype(jnp.float32)[..., None]
        f1 = ((xi >= 1) & (xi < 255)).astype(jnp.float32)[..., None]
        r0 = lut_ref[i, 0]
        r1 = lut_ref[i, 1]
        r2 = lut_ref[i, 2]
        out_ref[i] = r2 + f0 * (r0 - r2) + f1 * (r1 - r2)


def _tc_expand(x3, lut):
    return pl.pallas_call(
        _tc_expand_body,
        grid=(B // _BBLK,),
        in_specs=[
            pl.BlockSpec((_BBLK, NV, DEPTH), lambda g: (g, 0, 0)),
            pl.BlockSpec((NV, _LR, DM), lambda g: (0, 0, 0)),
        ],
        out_specs=pl.BlockSpec((NV, _BBLK, DEPTH, DM), lambda g: (0, g, 0, 0)),
        out_shape=jax.ShapeDtypeStruct((NV, B, DEPTH, DM), jnp.float32),
    )(x3, lut)


def kernel(x, emb_tables, W, b, gamma, beta):
    lut = _lut(emb_tables, W,
               b.reshape(NV, 1, DM),
               gamma.reshape(NV, 1, DM),
               beta.reshape(NV, 1, DM))
    out = _sc_expand()(lut.reshape(NROWS, DM), x.astype(jnp.int32))
    return out.reshape(NV, B, DEPTH, DM)


# speedup vs baseline: 1.0903x; 1.0903x over previous
import functools

import jax
import jax.numpy as jnp
from jax import lax
from jax.experimental import pallas as pl
from jax.experimental.pallas import tpu as pltpu
from jax.experimental.pallas import tpu_sc as plsc

NV = 26
DEPTH = 32
DE = 7
DM = 128
B = 1024
NTOK = NV * B * DEPTH
_LR = 8
NROWS = _LR * NV

_NC = 2
_NS = 16
_NW = _NC * _NS
_RPW = NV * B // _NW
_RS = 4
_S = _RS * DEPTH
_STEPS = _RPW // _RS
_NB = 4


def _lut_body(emb_ref, w_ref, b_ref, g_ref, bt_ref, lut_ref):
    rowmask = (lax.broadcasted_iota(jnp.int32, (_LR, 1), 0) < 2).astype(jnp.float32)
    for i in range(NV):
        e = jnp.concatenate(
            [emb_ref[i], jnp.zeros((_LR - 3, DE), jnp.float32)]) * rowmask
        h = lax.dot_general(e, w_ref[i], (((1,), (0,)), ((), ())),
                            preferred_element_type=jnp.float32)
        h = h + b_ref[i]
        mu = jnp.mean(h, axis=-1, keepdims=True)
        var = jnp.mean((h - mu) ** 2, axis=-1, keepdims=True)
        lut_ref[i] = (h - mu) * lax.rsqrt(var + 1e-5) * g_ref[i] + bt_ref[i]


def _lut(emb_tables, W, b3, g3, bt3):
    return pl.pallas_call(
        _lut_body,
        out_shape=jax.ShapeDtypeStruct((NV, _LR, DM), jnp.float32),
    )(emb_tables, W, b3, g3, bt3)


def _sc_body(lut_hbm, x_hbm, out_hbm, lut_s,
             x_v, idx_v, rows_v, sem_x, sem_g, sem_o):
    sid = lax.axis_index("s")
    wid = sid * _NC + lax.axis_index("c")
    base_r = wid * _RPW

    @pl.when(sid == 0)
    def _():
        pltpu.sync_copy(lut_hbm, lut_s)
    plsc.subcore_barrier()

    def fire_x(s, j):
        r0 = base_r + s * _RS
        pltpu.async_copy(x_hbm.at[pl.ds(r0 % B, _RS), pl.ds(r0 // B, 1)],
                         x_v[j], sem_x[j])

    def wait_x(j):
        pltpu.make_async_copy(x_hbm.at[pl.ds(0, _RS), pl.ds(0, 1)],
                              x_v[j], sem_x[j]).wait()

    def wait_o(j):
        pltpu.make_async_copy(rows_v[j], out_hbm.at[pl.ds(0, _S)],
                              sem_o[j]).wait()

    for j in range(_NB):
        fire_x(j, j)

    def step(k, carry):
        descs = []
        for j in range(_NB):
            s = k * _NB + j
            r0 = base_r + s * _RS
            rowb = _LR * (r0 // B)
            wait_x(j)

            @pl.when(k >= 1)
            def _():
                wait_o(j)

            for g in range(_S // 16):
                xv = x_v[j][g // 2, 0, pl.ds((g % 2) * 16, 16)]
                c = jnp.where(xv < 255, jnp.minimum(xv, 1), 2)
                idx_v[j][pl.ds(g * 16, 16)] = rowb + c

            descs.append(pltpu.async_copy(lut_s.at[idx_v[j]], rows_v[j],
                                          sem_g[j]))

            @pl.when(k < (_STEPS // _NB) - 1)
            def _():
                fire_x(s + _NB, j)

        for j in range(_NB):
            s = k * _NB + j
            r0 = base_r + s * _RS
            descs[j].wait()
            pltpu.async_copy(rows_v[j], out_hbm.at[pl.ds(r0 * DEPTH, _S)],
                             sem_o[j])
        return carry

    lax.fori_loop(0, _STEPS // _NB, step, 0)
    for j in range(_NB):
        wait_o(j)


@functools.cache
def _sc_expand():
    return functools.partial(
        pl.kernel,
        out_type=jax.ShapeDtypeStruct((NTOK, DM), jnp.float32),
        mesh=plsc.VectorSubcoreMesh(core_axis_name="c", subcore_axis_name="s"),
        scratch_types=[
            pltpu.VMEM_SHARED((NROWS, DM), jnp.float32),
            [pltpu.VMEM((_RS, 1, DEPTH), jnp.int32) for _ in range(_NB)],
            [pltpu.VMEM((_S,), jnp.int32) for _ in range(_NB)],
            [pltpu.VMEM((_S, DM), jnp.float32) for _ in range(_NB)],
            [pltpu.SemaphoreType.DMA for _ in range(_NB)],
            [pltpu.SemaphoreType.DMA for _ in range(_NB)],
            [pltpu.SemaphoreType.DMA for _ in range(_NB)],
        ],
    )(_sc_body)


def kernel(x, emb_tables, W, b, gamma, beta):
    lut = _lut(emb_tables, W,
               b.reshape(NV, 1, DM),
               gamma.reshape(NV, 1, DM),
               beta.reshape(NV, 1, DM))
    out = _sc_expand()(lut.reshape(NROWS, DM), x.astype(jnp.int32))
    return out.reshape(NV, B, DEPTH, DM)


# speedup vs baseline: 1.1787x; 1.0811x over previous
import functools

import jax
import jax.numpy as jnp
from jax import lax
from jax.experimental import pallas as pl
from jax.experimental.pallas import tpu as pltpu
from jax.experimental.pallas import tpu_sc as plsc

NV = 26
DEPTH = 32
DE = 7
DM = 128
B = 1024
NTOK = NV * B * DEPTH
_LR = 8
NROWS = _LR * NV

_NC = 2
_NS = 16
_NW = _NC * _NS
_RPW = NV * B // _NW
_RS = 2
_S = _RS * DEPTH
_STEPS = _RPW // _RS
_NB = 8


def _lut_body(emb_ref, w_ref, b_ref, g_ref, bt_ref, lut_ref):
    rowmask = (lax.broadcasted_iota(jnp.int32, (_LR, 1), 0) < 2).astype(jnp.float32)
    for i in range(NV):
        e = jnp.concatenate(
            [emb_ref[i], jnp.zeros((_LR - 3, DE), jnp.float32)]) * rowmask
        h = lax.dot_general(e, w_ref[i], (((1,), (0,)), ((), ())),
                            preferred_element_type=jnp.float32)
        h = h + b_ref[i]
        mu = jnp.mean(h, axis=-1, keepdims=True)
        var = jnp.mean((h - mu) ** 2, axis=-1, keepdims=True)
        lut_ref[i] = (h - mu) * lax.rsqrt(var + 1e-5) * g_ref[i] + bt_ref[i]


def _lut(emb_tables, W, b3, g3, bt3):
    return pl.pallas_call(
        _lut_body,
        out_shape=jax.ShapeDtypeStruct((NV, _LR, DM), jnp.float32),
    )(emb_tables, W, b3, g3, bt3)


def _sc_body(lut_hbm, x_hbm, out_hbm, lut_s,
             x_v, idx_v, rows_v, sem_x, sem_g, sem_o):
    sid = lax.axis_index("s")
    wid = sid * _NC + lax.axis_index("c")
    base_r = wid * _RPW

    @pl.when(sid == 0)
    def _():
        pltpu.sync_copy(lut_hbm, lut_s)
    plsc.subcore_barrier()

    def fire_x(s, j):
        r0 = base_r + s * _RS
        pltpu.async_copy(x_hbm.at[pl.ds(r0 % B, _RS), pl.ds(r0 // B, 1)],
                         x_v[j], sem_x[j])

    def wait_x(j):
        pltpu.make_async_copy(x_hbm.at[pl.ds(0, _RS), pl.ds(0, 1)],
                              x_v[j], sem_x[j]).wait()

    def wait_o(j):
        pltpu.make_async_copy(rows_v[j], out_hbm.at[pl.ds(0, _S)],
                              sem_o[j]).wait()

    for j in range(_NB):
        fire_x(j, j)

    def step(k, carry):
        descs = []
        for j in range(_NB):
            s = k * _NB + j
            r0 = base_r + s * _RS
            rowb = _LR * (r0 // B)
            wait_x(j)

            @pl.when(k >= 1)
            def _():
                wait_o(j)

            for g in range(_S // 16):
                xv = x_v[j][g // 2, 0, pl.ds((g % 2) * 16, 16)]
                c = jnp.where(xv < 255, jnp.minimum(xv, 1), 2)
                idx_v[j][pl.ds(g * 16, 16)] = rowb + c

            descs.append(pltpu.async_copy(lut_s.at[idx_v[j]], rows_v[j],
                                          sem_g[j]))

            @pl.when(k < (_STEPS // _NB) - 1)
            def _():
                fire_x(s + _NB, j)

        for j in range(_NB):
            s = k * _NB + j
            r0 = base_r + s * _RS
            descs[j].wait()
            pltpu.async_copy(rows_v[j], out_hbm.at[pl.ds(r0 * DEPTH, _S)],
                             sem_o[j])
        return carry

    lax.fori_loop(0, _STEPS // _NB, step, 0)
    for j in range(_NB):
        wait_o(j)


@functools.cache
def _sc_expand():
    return functools.partial(
        pl.kernel,
        out_type=jax.ShapeDtypeStruct((NTOK, DM), jnp.float32),
        mesh=plsc.VectorSubcoreMesh(core_axis_name="c", subcore_axis_name="s"),
        scratch_types=[
            pltpu.VMEM_SHARED((NROWS, DM), jnp.float32),
            [pltpu.VMEM((_RS, 1, DEPTH), jnp.int32) for _ in range(_NB)],
            [pltpu.VMEM((_S,), jnp.int32) for _ in range(_NB)],
            [pltpu.VMEM((_S, DM), jnp.float32) for _ in range(_NB)],
            [pltpu.SemaphoreType.DMA for _ in range(_NB)],
            [pltpu.SemaphoreType.DMA for _ in range(_NB)],
            [pltpu.SemaphoreType.DMA for _ in range(_NB)],
        ],
    )(_sc_body)


def kernel(x, emb_tables, W, b, gamma, beta):
    lut = _lut(emb_tables, W,
               b.reshape(NV, 1, DM),
               gamma.reshape(NV, 1, DM),
               beta.reshape(NV, 1, DM))
    out = _sc_expand()(lut.reshape(NROWS, DM), x.astype(jnp.int32))
    return out.reshape(NV, B, DEPTH, DM)


# speedup vs baseline: 1.2236x; 1.0381x over previous
import functools

import jax
import jax.numpy as jnp
from jax import lax
from jax.experimental import pallas as pl
from jax.experimental.pallas import tpu as pltpu
from jax.experimental.pallas import tpu_sc as plsc

NV = 26
DEPTH = 32
DE = 7
DM = 128
B = 1024
NTOK = NV * B * DEPTH
_LR = 8
NROWS = _LR * NV

_NC = 2
_NS = 16
_NW = _NC * _NS
_RPW = NV * B // _NW
_RS = 1
_S = _RS * DEPTH
_STEPS = _RPW // _RS
_NB = 8


def _lut_body(emb_ref, w_ref, b_ref, g_ref, bt_ref, lut_ref):
    rowmask = (lax.broadcasted_iota(jnp.int32, (_LR, 1), 0) < 2).astype(jnp.float32)
    for i in range(NV):
        e = jnp.concatenate(
            [emb_ref[i], jnp.zeros((_LR - 3, DE), jnp.float32)]) * rowmask
        h = lax.dot_general(e, w_ref[i], (((1,), (0,)), ((), ())),
                            preferred_element_type=jnp.float32)
        h = h + b_ref[i]
        mu = jnp.mean(h, axis=-1, keepdims=True)
        var = jnp.mean((h - mu) ** 2, axis=-1, keepdims=True)
        lut_ref[i] = (h - mu) * lax.rsqrt(var + 1e-5) * g_ref[i] + bt_ref[i]


def _lut(emb_tables, W, b3, g3, bt3):
    return pl.pallas_call(
        _lut_body,
        out_shape=jax.ShapeDtypeStruct((NV, _LR, DM), jnp.float32),
    )(emb_tables, W, b3, g3, bt3)


def _sc_body(lut_hbm, x_hbm, out_hbm, lut_s,
             x_v, idx_v, rows_v, sem_x, sem_g, sem_o):
    sid = lax.axis_index("s")
    wid = sid * _NC + lax.axis_index("c")
    base_r = wid * _RPW

    @pl.when(sid == 0)
    def _():
        pltpu.sync_copy(lut_hbm, lut_s)
    plsc.subcore_barrier()

    def fire_x(s, j):
        r0 = base_r + s * _RS
        pltpu.async_copy(x_hbm.at[pl.ds(r0 % B, _RS), pl.ds(r0 // B, 1)],
                         x_v[j], sem_x[j])

    def wait_x(j):
        pltpu.make_async_copy(x_hbm.at[pl.ds(0, _RS), pl.ds(0, 1)],
                              x_v[j], sem_x[j]).wait()

    def wait_o(j):
        pltpu.make_async_copy(rows_v[j], out_hbm.at[pl.ds(0, _S)],
                              sem_o[j]).wait()

    for j in range(_NB):
        fire_x(j, j)

    def step(k, carry):
        descs = []
        for j in range(_NB):
            s = k * _NB + j
            r0 = base_r + s * _RS
            rowb = _LR * (r0 // B)
            wait_x(j)

            @pl.when(k >= 1)
            def _():
                wait_o(j)

            for g in range(_S // 16):
                xv = x_v[j][g // 2, 0, pl.ds((g % 2) * 16, 16)]
                c = jnp.where(xv < 255, jnp.minimum(xv, 1), 2)
                idx_v[j][pl.ds(g * 16, 16)] = rowb + c

            descs.append(pltpu.async_copy(lut_s.at[idx_v[j]], rows_v[j],
                                          sem_g[j]))

            @pl.when(k < (_STEPS // _NB) - 1)
            def _():
                fire_x(s + _NB, j)

        for j in range(_NB):
            s = k * _NB + j
            r0 = base_r + s * _RS
            descs[j].wait()
            pltpu.async_copy(rows_v[j], out_hbm.at[pl.ds(r0 * DEPTH, _S)],
                             sem_o[j])
        return carry

    lax.fori_loop(0, _STEPS // _NB, step, 0)
    for j in range(_NB):
        wait_o(j)


@functools.cache
def _sc_expand():
    return functools.partial(
        pl.kernel,
        out_type=jax.ShapeDtypeStruct((NTOK, DM), jnp.float32),
        mesh=plsc.VectorSubcoreMesh(core_axis_name="c", subcore_axis_name="s"),
        scratch_types=[
            pltpu.VMEM_SHARED((NROWS, DM), jnp.float32),
            [pltpu.VMEM((_RS, 1, DEPTH), jnp.int32) for _ in range(_NB)],
            [pltpu.VMEM((_S,), jnp.int32) for _ in range(_NB)],
            [pltpu.VMEM((_S, DM), jnp.float32) for _ in range(_NB)],
            [pltpu.SemaphoreType.DMA for _ in range(_NB)],
            [pltpu.SemaphoreType.DMA for _ in range(_NB)],
            [pltpu.SemaphoreType.DMA for _ in range(_NB)],
        ],
    )(_sc_body)


def kernel(x, emb_tables, W, b, gamma, beta):
    lut = _lut(emb_tables, W,
               b.reshape(NV, 1, DM),
               gamma.reshape(NV, 1, DM),
               beta.reshape(NV, 1, DM))
    out = _sc_expand()(lut.reshape(NROWS, DM), x.astype(jnp.int32))
    return out.reshape(NV, B, DEPTH, DM)
